# baseline (speedup 1.0000x reference)
import jax
import jax.numpy as jnp
from jax import lax
from jax.experimental import pallas as pl
from jax.experimental.pallas import tpu as pltpu
from jax.experimental.pallas import tpu_sc as plsc

E = 8
K = 2
D = 1024
F = 4096
T = 2048
TM = 128
TN = 1024
TND = 512
TMO = 128
P = T * K + E * TM
NI = P // TM
NJ = F // TN
ND2 = D // TND
NTO = T // TMO

NC = 2
NS = 16
NW = NC * NS
GCH = (P // NW) // 4
assert P // NW == 4 * GCH and GCH % 8 == 0
CCH = 32


def _routing_kernel(g_ref, dest_ref, tw_ref, pend_ref):
    logits = g_ref[...]
    cols = jax.lax.broadcasted_iota(jnp.int32, (T, E), 1)
    l1 = jnp.max(logits, axis=-1, keepdims=True)
    i1 = jnp.min(jnp.where(logits == l1, cols, E), axis=-1, keepdims=True)
    masked = jnp.where(cols == i1, -jnp.inf, logits)
    l2 = jnp.max(masked, axis=-1, keepdims=True)
    i2 = jnp.min(jnp.where(masked == l2, cols, E), axis=-1, keepdims=True)
    e2 = jnp.exp(l2 - l1)
    denom = 1.0 + e2
    tw_ref[...] = jnp.concatenate([1.0 / denom, e2 / denom], axis=1)

    oh0 = (cols == i1).astype(jnp.float32)
    oh1 = (cols == i2).astype(jnp.float32)
    both = oh0 + oh1
    counts = jnp.sum(both, axis=0, keepdims=True)
    padded = jnp.ceil(counts / TM) * TM
    ea = jax.lax.broadcasted_iota(jnp.int32, (E, E), 0)
    eb = jax.lax.broadcasted_iota(jnp.int32, (E, E), 1)
    pend = jnp.dot(padded, (ea <= eb).astype(jnp.float32),
                   preferred_element_type=jnp.float32)
    pstart = pend - padded
    ta = jax.lax.broadcasted_iota(jnp.int32, (T, T), 0)
    tb = jax.lax.broadcasted_iota(jnp.int32, (T, T), 1)
    ltri = (tb < ta).astype(jnp.float32)
    csum = jnp.dot(ltri, both, preferred_element_type=jnp.float32)
    d0 = jnp.sum((pstart + csum) * oh0, axis=-1, keepdims=True)
    d1 = jnp.sum((pstart + csum) * oh1, axis=-1, keepdims=True)
    dest_ref[...] = jnp.concatenate([d0, d1], axis=1).astype(jnp.int32)
    pend_ref[...] = pend.astype(jnp.int32)


def _sc_gather_kernel(x_hbm, idx_hbm, out_hbm,
                      i0, i1, b0, b1, s0, s1):
    wid = lax.axis_index("s") * NC + lax.axis_index("c")
    base = wid * (P // NW)
    nch = P // NW // GCH
    ib = [i0, i1]
    bb = [b0, b1]
    ss = [s0, s1]
    pltpu.sync_copy(idx_hbm.at[pl.ds(base, GCH)], i0)
    cps = [pltpu.async_copy(x_hbm.at[i0], b0, s0)]
    for c in range(nch):
        if c + 1 < nch:
            k = (c + 1) % 2
            pltpu.sync_copy(idx_hbm.at[pl.ds(base + (c + 1) * GCH, GCH)],
                            ib[k])
            cps.append(pltpu.async_copy(x_hbm.at[ib[k]], bb[k], ss[k]))
        cps[c].wait()
        pltpu.sync_copy(bb[c % 2], out_hbm.at[pl.ds(base + c * GCH, GCH)])


def _sc_combine_kernel(y_hbm, p0_hbm, p1_hbm, out_hbm,
                       i0_v, i1_v, b0, b1, sem0, sem1):
    wid = lax.axis_index("s") * NC + lax.axis_index("c")
    base = wid * (T // NW)
    for c in range(T // NW // CCH):
        off = base + c * CCH
        pltpu.sync_copy(p0_hbm.at[pl.ds(off, CCH)], i0_v)
        pltpu.sync_copy(p1_hbm.at[pl.ds(off, CCH)], i1_v)
        cp0 = pltpu.async_copy(y_hbm.at[i0_v], b0, sem0)
        cp1 = pltpu.async_copy(y_hbm.at[i1_v], b1, sem1)
        cp0.wait()
        cp1.wait()
        for r in range(CCH):
            def body(k, _, r=r):
                b0[r, pl.ds(k * 16, 16)] = (b0[r, pl.ds(k * 16, 16)]
                                            + b1[r, pl.ds(k * 16, 16)])
                return 0

            lax.fori_loop(0, D // 16, body, 0)
        pltpu.sync_copy(b0, out_hbm.at[pl.ds(off, CCH)])


def _gemm1_kernel(tile_expert_ref, tile_valid_ref,
                  xg_ref, w1g_ref, w1u_ref, rw_ref, h_ref):
    i = pl.program_id(1)
    valid = tile_valid_ref[i] == 1

    @pl.when(valid)
    def _compute():
        xx = xg_ref[pl.ds(i * TM, TM), :].astype(jnp.bfloat16)
        wg = w1g_ref[0].astype(jnp.bfloat16)
        wu = w1u_ref[0].astype(jnp.bfloat16)
        g = jnp.dot(xx, wg, preferred_element_type=jnp.float32)
        u = jnp.dot(xx, wu, preferred_element_type=jnp.float32)
        h_ref[...] = ((g * jax.lax.logistic(g)) * u
                      * rw_ref[...]).astype(jnp.bfloat16)


def _gemm2_kernel(tile_expert_ref, tile_valid_ref, h_ref, w2_ref, y_ref):
    i = pl.program_id(1)
    valid = tile_valid_ref[i] == 1

    @pl.when(valid)
    def _compute():
        y_ref[...] = jnp.dot(h_ref[...],
                             w2_ref[0].astype(jnp.bfloat16),
                             preferred_element_type=jnp.float32)


def kernel(hidden_states, gating_output, w1, w2):
    dest, tw, pend = pl.pallas_call(
        _routing_kernel,
        out_shape=(
            jax.ShapeDtypeStruct((T, K), jnp.int32),
            jax.ShapeDtypeStruct((T, K), jnp.float32),
            jax.ShapeDtypeStruct((1, E), jnp.int32),
        ),
    )(gating_output)

    pos = dest.reshape(-1)
    row_token = jnp.zeros((P,), jnp.int32).at[pos].set(
        jnp.arange(T * K, dtype=jnp.int32) // K)
    row_w = jnp.zeros((P,), jnp.float32).at[pos].set(tw.reshape(-1))
    pend1 = pend.reshape(-1)
    tile_start = jnp.arange(NI, dtype=jnp.int32) * TM
    tile_expert = jnp.minimum(
        jnp.sum((tile_start[:, None] >= pend1[None, :]).astype(jnp.int32),
                axis=1), E - 1).astype(jnp.int32)
    tile_valid = (tile_start < pend1[-1]).astype(jnp.int32)

    xg = pl.kernel(
        _sc_gather_kernel,
        out_type=jax.ShapeDtypeStruct((P, D), jnp.float32),
        mesh=plsc.VectorSubcoreMesh(core_axis_name="c", subcore_axis_name="s"),
        scratch_types=[
            pltpu.VMEM((GCH,), jnp.int32),
            pltpu.VMEM((GCH,), jnp.int32),
            pltpu.VMEM((GCH, D), jnp.float32),
            pltpu.VMEM((GCH, D), jnp.float32),
            pltpu.SemaphoreType.DMA,
            pltpu.SemaphoreType.DMA,
        ],
    )(hidden_states, row_token)

    h = pl.pallas_call(
        _gemm1_kernel,
        grid_spec=pltpu.PrefetchScalarGridSpec(
            num_scalar_prefetch=2,
            grid=(NJ, NI),
            in_specs=[
                pl.BlockSpec((P, D), lambda j, i, *_: (0, 0)),
                pl.BlockSpec((1, D, TN), lambda j, i, te, tv: (te[i], 0, j)),
                pl.BlockSpec((1, D, TN), lambda j, i, te, tv: (te[i], 0, j + NJ)),
                pl.BlockSpec((TM, 1), lambda j, i, *_: (i, 0)),
            ],
            out_specs=pl.BlockSpec((TM, TN), lambda j, i, *_: (i, j)),
        ),
        out_shape=jax.ShapeDtypeStruct((P, F), jnp.bfloat16),
    )(tile_expert, tile_valid, xg, w1, w1, row_w.reshape(P, 1))

    y = pl.pallas_call(
        _gemm2_kernel,
        grid_spec=pltpu.PrefetchScalarGridSpec(
            num_scalar_prefetch=2,
            grid=(ND2, NI),
            in_specs=[
                pl.BlockSpec((TM, F), lambda jd, i, *_: (i, 0)),
                pl.BlockSpec((1, F, TND), lambda jd, i, te, tv: (te[i], 0, jd)),
            ],
            out_specs=pl.BlockSpec((TM, TND), lambda jd, i, *_: (i, jd)),
        ),
        out_shape=jax.ShapeDtypeStruct((P, D), jnp.float32),
    )(tile_expert, tile_valid, h, w2)

    out = pl.kernel(
        _sc_combine_kernel,
        out_type=jax.ShapeDtypeStruct((T, D), jnp.float32),
        mesh=plsc.VectorSubcoreMesh(core_axis_name="c", subcore_axis_name="s"),
        scratch_types=[
            pltpu.VMEM((CCH,), jnp.int32),
            pltpu.VMEM((CCH,), jnp.int32),
            pltpu.VMEM((CCH, D), jnp.float32),
            pltpu.VMEM((CCH, D), jnp.float32),
            pltpu.SemaphoreType.DMA,
            pltpu.SemaphoreType.DMA,
        ],
    )(y, dest[:, 0] + 0, dest[:, 1] + 0)
    return out


# speedup vs baseline: 1.1607x; 1.1607x over previous
import jax
import jax.numpy as jnp
from jax import lax
from jax.experimental import pallas as pl
from jax.experimental.pallas import tpu as pltpu
from jax.experimental.pallas import tpu_sc as plsc

E = 8
K = 2
D = 1024
F = 4096
T = 2048
TM = 128
TN = 1024
TND = 512
TMO = 128
P = T * K + E * TM
NI = P // TM
NJ = F // TN
ND2 = D // TND
NTO = T // TMO

NC = 2
NS = 16
NW = NC * NS
TPW = T // NW
CCH = 32


def _routing_kernel(g_ref, dest_ref, tw_ref, pend_ref):
    logits = g_ref[...]
    cols = jax.lax.broadcasted_iota(jnp.int32, (T, E), 1)
    l1 = jnp.max(logits, axis=-1, keepdims=True)
    i1 = jnp.min(jnp.where(logits == l1, cols, E), axis=-1, keepdims=True)
    masked = jnp.where(cols == i1, -jnp.inf, logits)
    l2 = jnp.max(masked, axis=-1, keepdims=True)
    i2 = jnp.min(jnp.where(masked == l2, cols, E), axis=-1, keepdims=True)
    e2 = jnp.exp(l2 - l1)
    denom = 1.0 + e2
    tw_ref[...] = jnp.concatenate([1.0 / denom, e2 / denom], axis=1)

    oh0 = (cols == i1).astype(jnp.float32)
    oh1 = (cols == i2).astype(jnp.float32)
    both = oh0 + oh1
    counts = jnp.sum(both, axis=0, keepdims=True)
    padded = jnp.ceil(counts / TM) * TM
    ea = jax.lax.broadcasted_iota(jnp.int32, (E, E), 0)
    eb = jax.lax.broadcasted_iota(jnp.int32, (E, E), 1)
    pend = jnp.dot(padded, (ea <= eb).astype(jnp.float32),
                   preferred_element_type=jnp.float32)
    pstart = pend - padded
    ta = jax.lax.broadcasted_iota(jnp.int32, (T, T), 0)
    tb = jax.lax.broadcasted_iota(jnp.int32, (T, T), 1)
    ltri = (tb < ta).astype(jnp.float32)
    csum = jnp.dot(ltri, both, preferred_element_type=jnp.float32)
    d0 = jnp.sum((pstart + csum) * oh0, axis=-1, keepdims=True)
    d1 = jnp.sum((pstart + csum) * oh1, axis=-1, keepdims=True)
    dest_ref[...] = jnp.concatenate([d0, d1], axis=1).astype(jnp.int32)
    pend_ref[...] = pend.astype(jnp.int32)


def _sc_dispatch_kernel(x_hbm, d0_hbm, d1_hbm, out_hbm,
                        i0_v, i1_v, rows_v, s0, s1):
    wid = lax.axis_index("s") * NC + lax.axis_index("c")
    base = wid * TPW
    pltpu.sync_copy(d0_hbm.at[pl.ds(base, TPW)], i0_v)
    pltpu.sync_copy(d1_hbm.at[pl.ds(base, TPW)], i1_v)
    pltpu.sync_copy(x_hbm.at[pl.ds(base, TPW)], rows_v)
    cp0 = pltpu.async_copy(rows_v, out_hbm.at[i0_v], s0)
    cp1 = pltpu.async_copy(rows_v, out_hbm.at[i1_v], s1)
    cp0.wait()
    cp1.wait()


def _sc_combine_kernel(y_hbm, p0_hbm, p1_hbm, out_hbm,
                       i0_v, i1_v, b0, b1, sem0, sem1):
    wid = lax.axis_index("s") * NC + lax.axis_index("c")
    base = wid * (T // NW)
    for c in range(T // NW // CCH):
        off = base + c * CCH
        pltpu.sync_copy(p0_hbm.at[pl.ds(off, CCH)], i0_v)
        pltpu.sync_copy(p1_hbm.at[pl.ds(off, CCH)], i1_v)
        cp0 = pltpu.async_copy(y_hbm.at[i0_v], b0, sem0)
        cp1 = pltpu.async_copy(y_hbm.at[i1_v], b1, sem1)
        cp0.wait()
        cp1.wait()
        for r in range(CCH):
            def body(k, _, r=r):
                b0[r, pl.ds(k * 16, 16)] = (b0[r, pl.ds(k * 16, 16)]
                                            + b1[r, pl.ds(k * 16, 16)])
                return 0

            lax.fori_loop(0, D // 16, body, 0)
        pltpu.sync_copy(b0, out_hbm.at[pl.ds(off, CCH)])


def _gemm1_kernel(tile_expert_ref, tile_valid_ref,
                  xg_ref, w1g_ref, w1u_ref, rw_ref, h_ref):
    i = pl.program_id(1)
    valid = tile_valid_ref[i] == 1

    @pl.when(valid)
    def _compute():
        xx = xg_ref[pl.ds(i * TM, TM), :].astype(jnp.bfloat16)
        wg = w1g_ref[0].astype(jnp.bfloat16)
        wu = w1u_ref[0].astype(jnp.bfloat16)
        g = jnp.dot(xx, wg, preferred_element_type=jnp.float32)
        u = jnp.dot(xx, wu, preferred_element_type=jnp.float32)
        h_ref[...] = ((g * jax.lax.logistic(g)) * u
                      * rw_ref[...]).astype(jnp.bfloat16)


def _gemm2_kernel(tile_expert_ref, tile_valid_ref, h_ref, w2_ref, y_ref):
    i = pl.program_id(1)
    valid = tile_valid_ref[i] == 1

    @pl.when(valid)
    def _compute():
        y_ref[...] = jnp.dot(h_ref[...],
                             w2_ref[0].astype(jnp.bfloat16),
                             preferred_element_type=jnp.float32)


def kernel(hidden_states, gating_output, w1, w2):
    dest, tw, pend = pl.pallas_call(
        _routing_kernel,
        out_shape=(
            jax.ShapeDtypeStruct((T, K), jnp.int32),
            jax.ShapeDtypeStruct((T, K), jnp.float32),
            jax.ShapeDtypeStruct((1, E), jnp.int32),
        ),
    )(gating_output)

    pos = dest.reshape(-1)
    row_w = jnp.zeros((P,), jnp.float32).at[pos].set(tw.reshape(-1))
    pend1 = pend.reshape(-1)
    tile_start = jnp.arange(NI, dtype=jnp.int32) * TM
    tile_expert = jnp.minimum(
        jnp.sum((tile_start[:, None] >= pend1[None, :]).astype(jnp.int32),
                axis=1), E - 1).astype(jnp.int32)
    tile_valid = (tile_start < pend1[-1]).astype(jnp.int32)

    xg = pl.kernel(
        _sc_dispatch_kernel,
        out_type=jax.ShapeDtypeStruct((P, D), jnp.float32),
        mesh=plsc.VectorSubcoreMesh(core_axis_name="c", subcore_axis_name="s"),
        scratch_types=[
            pltpu.VMEM((TPW,), jnp.int32),
            pltpu.VMEM((TPW,), jnp.int32),
            pltpu.VMEM((TPW, D), jnp.float32),
            pltpu.SemaphoreType.DMA,
            pltpu.SemaphoreType.DMA,
        ],
    )(hidden_states, dest[:, 0] + 0, dest[:, 1] + 0)

    h = pl.pallas_call(
        _gemm1_kernel,
        grid_spec=pltpu.PrefetchScalarGridSpec(
            num_scalar_prefetch=2,
            grid=(NJ, NI),
            in_specs=[
                pl.BlockSpec((P, D), lambda j, i, *_: (0, 0)),
                pl.BlockSpec((1, D, TN), lambda j, i, te, tv: (te[i], 0, j)),
                pl.BlockSpec((1, D, TN), lambda j, i, te, tv: (te[i], 0, j + NJ)),
                pl.BlockSpec((TM, 1), lambda j, i, *_: (i, 0)),
            ],
            out_specs=pl.BlockSpec((TM, TN), lambda j, i, *_: (i, j)),
        ),
        out_shape=jax.ShapeDtypeStruct((P, F), jnp.bfloat16),
    )(tile_expert, tile_valid, xg, w1, w1, row_w.reshape(P, 1))

    y = pl.pallas_call(
        _gemm2_kernel,
        grid_spec=pltpu.PrefetchScalarGridSpec(
            num_scalar_prefetch=2,
            grid=(ND2, NI),
            in_specs=[
                pl.BlockSpec((TM, F), lambda jd, i, *_: (i, 0)),
                pl.BlockSpec((1, F, TND), lambda jd, i, te, tv: (te[i], 0, jd)),
            ],
            out_specs=pl.BlockSpec((TM, TND), lambda jd, i, *_: (i, jd)),
        ),
        out_shape=jax.ShapeDtypeStruct((P, D), jnp.float32),
    )(tile_expert, tile_valid, h, w2)

    out = pl.kernel(
        _sc_combine_kernel,
        out_type=jax.ShapeDtypeStruct((T, D), jnp.float32),
        mesh=plsc.VectorSubcoreMesh(core_axis_name="c", subcore_axis_name="s"),
        scratch_types=[
            pltpu.VMEM((CCH,), jnp.int32),
            pltpu.VMEM((CCH,), jnp.int32),
            pltpu.VMEM((CCH, D), jnp.float32),
            pltpu.VMEM((CCH, D), jnp.float32),
            pltpu.SemaphoreType.DMA,
            pltpu.SemaphoreType.DMA,
        ],
    )(y, dest[:, 0] + 0, dest[:, 1] + 0)
    return out
